# baseline (speedup 1.0000x reference)
import functools

import jax
import jax.numpy as jnp
import numpy as np
from jax import lax
from jax.experimental import pallas as pl
from jax.experimental.pallas import tpu as pltpu
from jax.experimental.pallas import tpu_sc as plsc

RADIUS = 0.2
K = 32
R2 = np.float32(RADIUS * RADIUS)

B = 4
N = 8192
NQ = 1024
C = 128

NC = 2
NSUB = 16
L = 16
NW = NC * NSUB

WPB = NW // B
QPW = NQ // WPB
CHUNK = QPW
UNROLL = 8

SPG = K // WPB
NSTR = SPG * (NQ // CHUNK)


def _transpose_feat_body(f_ref, o_ref):
    o_ref[...] = jnp.transpose(f_ref[...], (0, 2, 1))


def _feat_transpose(features):
    return pl.pallas_call(
        _transpose_feat_body,
        grid=(B, N // 512),
        in_specs=[pl.BlockSpec((1, C, 512), lambda b, j: (b, 0, j))],
        out_specs=pl.BlockSpec((1, 512, C), lambda b, j: (b, j, 0)),
        out_shape=jax.ShapeDtypeStruct((B, N, C), jnp.float32),
    )(features)


def _scan_body(xyzT, newT, gxyz_out, idx_out,
               xpl, ypl, zpl, qx, qy, qz, buf, idxb, gxs, gys, gzs):
    cid = lax.axis_index("c")
    sid = lax.axis_index("s")
    wid = cid * NSUB + sid
    b = wid // WPB
    wq = wid % WPB
    qbase = wq * QPW

    pltpu.sync_copy(xyzT.at[pl.ds((b * 3 + 0) * N, N)], xpl)
    pltpu.sync_copy(xyzT.at[pl.ds((b * 3 + 1) * N, N)], ypl)
    pltpu.sync_copy(xyzT.at[pl.ds((b * 3 + 2) * N, N)], zpl)
    pltpu.sync_copy(newT.at[pl.ds((b * 3 + 0) * NQ + qbase, QPW)], qx)
    pltpu.sync_copy(newT.at[pl.ds((b * 3 + 1) * NQ + qbase, QPW)], qy)
    pltpu.sync_copy(newT.at[pl.ds((b * 3 + 2) * NQ + qbase, QPW)], qz)

    iota = lax.iota(jnp.int32, L)
    zeros = jnp.zeros((L,), jnp.int32)
    boffs = b * N

    def per_query(qi, carry):
        qi16 = zeros + qi
        qxv = plsc.load_gather(qx, [qi16])
        qyv = plsc.load_gather(qy, [qi16])
        qzv = plsc.load_gather(qz, [qi16])

        def cond(jc):
            j, _, cnt_s = jc
            return jnp.logical_and(j < N, cnt_s < K)

        def body(jc):
            j, csum, _ = jc
            ms = []
            for u in range(UNROLL):
                o = u * L
                dx = xpl[pl.ds(j + o, L)] - qxv
                dy = ypl[pl.ds(j + o, L)] - qyv
                dz = zpl[pl.ds(j + o, L)] - qzv
                d2 = dx * dx + dy * dy + dz * dz
                ms.append(d2 <= R2)
            for u in range(UNROLL):
                m = ms[u]
                pos = csum + plsc.cumsum(m.astype(jnp.int32)) - 1
                plsc.store_scatter(buf, [pos], j + u * L + iota, mask=m)
                csum = csum + plsc.all_reduce_population_count(m)
            return j + UNROLL * L, csum, jnp.max(csum)

        _, _, cnt = lax.while_loop(
            cond, body, (jnp.int32(0), jnp.zeros((L,), jnp.int32), jnp.int32(0)))

        blo = buf[pl.ds(0, L)]
        bhi = buf[pl.ds(L, L)]
        firstv = plsc.load_gather(buf, [zeros])
        fv = jnp.where(cnt > 0, firstv, 0)
        ilo = jnp.where(iota < cnt, blo, fv)
        ihi = jnp.where(iota + L < cnt, bhi, fv)

        rlo = iota
        rhi = iota + L
        z16 = zeros
        plsc.store_scatter(idxb, [rlo, z16, qi16], ilo + boffs)
        plsc.store_scatter(idxb, [rhi, z16, qi16], ihi + boffs)
        plsc.store_scatter(gxs, [rlo, qi16], plsc.load_gather(xpl, [ilo]) - qxv)
        plsc.store_scatter(gxs, [rhi, qi16], plsc.load_gather(xpl, [ihi]) - qxv)
        plsc.store_scatter(gys, [rlo, qi16], plsc.load_gather(ypl, [ilo]) - qyv)
        plsc.store_scatter(gys, [rhi, qi16], plsc.load_gather(ypl, [ihi]) - qyv)
        plsc.store_scatter(gzs, [rlo, qi16], plsc.load_gather(zpl, [ilo]) - qzv)
        plsc.store_scatter(gzs, [rhi, qi16], plsc.load_gather(zpl, [ihi]) - qzv)
        return carry

    lax.fori_loop(0, QPW, per_query, 0)

    pltpu.sync_copy(gxs, gxyz_out.at[b, 0, :, pl.ds(qbase, QPW)])
    pltpu.sync_copy(gys, gxyz_out.at[b, 1, :, pl.ds(qbase, QPW)])
    pltpu.sync_copy(gzs, gxyz_out.at[b, 2, :, pl.ds(qbase, QPW)])

    pltpu.sync_copy(idxb, idx_out.at[b, :, pl.ds(wq, 1), :])


NSTR = SPG * (NQ // CHUNK)
QG = CHUNK // L


def _gather_body(featT, idx_in, gxyz_in, out4,
                 idxv, rows0, rows1, tbuf, gxv, sem0, sem1):
    cid = lax.axis_index("c")
    sid = lax.axis_index("s")
    wid = cid * NSUB + sid
    b = wid // WPB
    sg = wid % WPB

    pltpu.sync_copy(idx_in.at[b, pl.ds(sg * SPG, SPG), :, :], idxv)

    def copy_gx(si):
        s = sg * SPG + si
        pltpu.sync_copy(gxyz_in.at[b, :, pl.ds(s, 1), :], gxv)
        pltpu.sync_copy(gxv, out4.at[b, pl.ds(0, 3), pl.ds(s, 1), :])

    for si in range(SPG):
        copy_gx(si)

    iota = lax.iota(jnp.int32, L)
    zeros = jnp.zeros((L,), jnp.int32)
    qrows = [iota + qg * L for qg in range(QG)]

    def issue(t, rbuf, sem):
        si = t // WPB
        qw = t % WPB
        pltpu.async_copy(featT.at[idxv.at[si, qw]], rbuf, sem)

    def drain(rbuf, sem):
        pltpu.make_async_copy(featT.at[pl.ds(0, CHUNK), :], rbuf, sem).wait()

    def transpose_write(t, rbuf):
        si = t // WPB
        qw = t % WPB

        def col(c, carry):
            cvec = zeros + c
            for qg in range(QG):
                v = plsc.load_gather(rbuf, [qrows[qg], cvec])
                tbuf[c, 0, pl.ds(qg * L, L)] = v
            return carry

        lax.fori_loop(0, C, col, 0)
        s = sg * SPG + si
        pltpu.sync_copy(
            tbuf,
            out4.at[b, pl.ds(3, C), pl.ds(s, 1), pl.ds(qw * CHUNK, CHUNK)])

    issue(0, rows0, sem0)

    def ring(g, carry):
        t0 = 2 * g
        t1 = 2 * g + 1
        issue(t1, rows1, sem1)
        drain(rows0, sem0)
        transpose_write(t0, rows0)

        @pl.when(g < NSTR // 2 - 1)
        def _():
            issue(t0 + 2, rows0, sem0)

        drain(rows1, sem1)
        transpose_write(t1, rows1)
        return carry

    lax.fori_loop(0, NSTR // 2, ring, 0)


def _make_calls():
    mesh = plsc.VectorSubcoreMesh(core_axis_name="c", subcore_axis_name="s")
    cp = pltpu.CompilerParams(needs_layout_passes=False)
    scan_call = functools.partial(
        pl.kernel,
        out_type=(
            jax.ShapeDtypeStruct((B, 3, K, NQ), jnp.float32),
            jax.ShapeDtypeStruct((B, K, WPB, QPW), jnp.int32),
        ),
        mesh=mesh,
        compiler_params=cp,
        scratch_types=[
            pltpu.VMEM((N,), jnp.float32),
            pltpu.VMEM((N,), jnp.float32),
            pltpu.VMEM((N,), jnp.float32),
            pltpu.VMEM((QPW,), jnp.float32),
            pltpu.VMEM((QPW,), jnp.float32),
            pltpu.VMEM((QPW,), jnp.float32),
            pltpu.VMEM((256,), jnp.int32),
            pltpu.VMEM((K, 1, QPW), jnp.int32),
            pltpu.VMEM((K, QPW), jnp.float32),
            pltpu.VMEM((K, QPW), jnp.float32),
            pltpu.VMEM((K, QPW), jnp.float32),
        ],
    )(_scan_body)
    gather_call = functools.partial(
        pl.kernel,
        out_type=jax.ShapeDtypeStruct((B, 3 + C, K, NQ), jnp.float32),
        mesh=mesh,
        compiler_params=cp,
        scratch_types=[
            pltpu.VMEM((SPG, WPB, QPW), jnp.int32),
            pltpu.VMEM((CHUNK, C), jnp.float32),
            pltpu.VMEM((CHUNK, C), jnp.float32),
            pltpu.VMEM((C, 1, CHUNK), jnp.float32),
            pltpu.VMEM((3, 1, NQ), jnp.float32),
            pltpu.SemaphoreType.DMA,
            pltpu.SemaphoreType.DMA,
        ],
    )(_gather_body)

    return scan_call, gather_call


_scan_call, _gather_call = _make_calls()


@jax.jit
def kernel(xyz, new_xyz, features):
    featT = _feat_transpose(features).reshape(B * N, C)
    xyzT = jnp.transpose(xyz, (0, 2, 1)).reshape(-1)
    newT = jnp.transpose(new_xyz, (0, 2, 1)).reshape(-1)
    gxyz_sc, idx = _scan_call(xyzT, newT)
    out4 = _gather_call(featT, idx, gxyz_sc)
    return jnp.transpose(out4, (0, 1, 3, 2))


# speedup vs baseline: 1.4737x; 1.4737x over previous
import functools

import jax
import jax.numpy as jnp
import numpy as np
from jax import lax
from jax.experimental import pallas as pl
from jax.experimental.pallas import tpu as pltpu
from jax.experimental.pallas import tpu_sc as plsc

RADIUS = 0.2
K = 32
R2 = np.float32(RADIUS * RADIUS)

B = 4
N = 8192
NQ = 1024
C = 128
W = 136

NC = 2
NSUB = 16
L = 16
NW = NC * NSUB

BPC = 2
WPB = NW // BPC
QPW = NQ // WPB
CHUNK = 128
NSTR = (K * QPW) // CHUNK
UNROLL = 8


def _table_body(f_ref, o_ref):
    o_ref[...] = jnp.transpose(f_ref[...], (0, 2, 1))


def _build_table(features):
    return pl.pallas_call(
        _table_body,
        grid=(B, N // 512),
        in_specs=[pl.BlockSpec((1, C, 512), lambda b, j: (b, 0, j))],
        out_specs=pl.BlockSpec((1, 512, C), lambda b, j: (b, j, 0)),
        out_shape=jax.ShapeDtypeStruct((B, N, C), jnp.float32),
    )(features)


def _make_sc_body(bbase):
    def _sc_body(xyzT, newT, tab, gxyz_out, gfeat_out,
                 xpl, ypl, zpl, qx, qy, qz, buf, idxb, gsta,
                 rows0, rows1, sem0, sem1, semg):
        cid = lax.axis_index("c")
        sid = lax.axis_index("s")
        wid = cid * NSUB + sid
        bl = wid // WPB
        b = bbase + bl
        qbase = (wid % WPB) * QPW

        pltpu.sync_copy(xyzT.at[pl.ds((b * 3 + 0) * N, N)], xpl)
        pltpu.sync_copy(xyzT.at[pl.ds((b * 3 + 1) * N, N)], ypl)
        pltpu.sync_copy(xyzT.at[pl.ds((b * 3 + 2) * N, N)], zpl)
        pltpu.sync_copy(newT.at[pl.ds((b * 3 + 0) * NQ + qbase, QPW)], qx)
        pltpu.sync_copy(newT.at[pl.ds((b * 3 + 1) * NQ + qbase, QPW)], qy)
        pltpu.sync_copy(newT.at[pl.ds((b * 3 + 2) * NQ + qbase, QPW)], qz)

        iota = lax.iota(jnp.int32, L)
        zeros = jnp.zeros((L,), jnp.int32)
        boffs = b * N

        def per_query(qi, carry):
            qi16 = zeros + qi
            qxv = plsc.load_gather(qx, [qi16])
            qyv = plsc.load_gather(qy, [qi16])
            qzv = plsc.load_gather(qz, [qi16])

            def cond(jc):
                j, _, cnt_s = jc
                return jnp.logical_and(j < N, cnt_s < K)

            def body(jc):
                j, csum, _ = jc
                ms = []
                for u in range(UNROLL):
                    o = u * L
                    dx = xpl[pl.ds(j + o, L)] - qxv
                    dy = ypl[pl.ds(j + o, L)] - qyv
                    dz = zpl[pl.ds(j + o, L)] - qzv
                    d2 = dx * dx + dy * dy + dz * dz
                    ms.append(d2 <= R2)
                for u in range(UNROLL):
                    m = ms[u]
                    pos = csum + plsc.cumsum(m.astype(jnp.int32)) - 1
                    plsc.store_scatter(buf, [pos], j + u * L + iota, mask=m)
                    csum = csum + plsc.all_reduce_population_count(m)
                return j + UNROLL * L, csum, jnp.max(csum)

            _, _, cnt = lax.while_loop(
                cond, body,
                (jnp.int32(0), jnp.zeros((L,), jnp.int32), jnp.int32(0)))

            blo = buf[pl.ds(0, L)]
            bhi = buf[pl.ds(L, L)]
            firstv = plsc.load_gather(buf, [zeros])
            fv = jnp.where(cnt > 0, firstv, 0)
            ilo = jnp.where(iota < cnt, blo, fv)
            ihi = jnp.where(iota + L < cnt, bhi, fv)

            plo = iota * QPW + qi
            phi = (iota + L) * QPW + qi
            plsc.store_scatter(idxb, [plo], ilo + boffs)
            plsc.store_scatter(idxb, [phi], ihi + boffs)
            KQ = K * QPW
            plsc.store_scatter(gsta, [plo], plsc.load_gather(xpl, [ilo]) - qxv)
            plsc.store_scatter(gsta, [phi], plsc.load_gather(xpl, [ihi]) - qxv)
            plsc.store_scatter(gsta, [KQ + plo], plsc.load_gather(ypl, [ilo]) - qyv)
            plsc.store_scatter(gsta, [KQ + phi], plsc.load_gather(ypl, [ihi]) - qyv)
            plsc.store_scatter(gsta, [2 * KQ + plo], plsc.load_gather(zpl, [ilo]) - qzv)
            plsc.store_scatter(gsta, [2 * KQ + phi], plsc.load_gather(zpl, [ihi]) - qzv)
            return carry

        lax.fori_loop(0, QPW, per_query, 0)

        for ch in range(3):
            for s in range(K):
                src = gsta.at[pl.ds((ch * K + s) * QPW, QPW)]
                dst = gxyz_out.at[
                    pl.ds(((bl * 3 + ch) * K + s) * NQ + qbase, QPW)]
                pltpu.async_copy(src, dst, semg)
        pltpu.make_async_copy(xyzT.at[pl.ds(0, 3 * K * QPW)], gsta, semg).wait()

        def issue(t, rbuf, sem):
            pltpu.async_copy(tab.at[idxb.at[pl.ds(t * CHUNK, CHUNK)]],
                             rbuf, sem)

        def drain(rbuf, sem):
            pltpu.make_async_copy(tab.at[pl.ds(0, CHUNK), :], rbuf, sem).wait()

        def writeback(t, rbuf):
            s = 2 * t
            r0 = (bl * K + s) * NQ + qbase
            r1 = (bl * K + s + 1) * NQ + qbase
            pltpu.sync_copy(rbuf.at[pl.ds(0, QPW), :],
                            gfeat_out.at[pl.ds(r0, QPW), :])
            pltpu.sync_copy(rbuf.at[pl.ds(QPW, QPW), :],
                            gfeat_out.at[pl.ds(r1, QPW), :])

        issue(0, rows0, sem0)

        def ring(g, carry):
            t0 = 2 * g
            t1 = 2 * g + 1
            issue(t1, rows1, sem1)
            drain(rows0, sem0)
            writeback(t0, rows0)

            @pl.when(g < NSTR // 2 - 1)
            def _():
                issue(t0 + 2, rows0, sem0)

            drain(rows1, sem1)
            writeback(t1, rows1)
            return carry

        lax.fori_loop(0, NSTR // 2, ring, 0)

    return _sc_body


def _make_calls():
    mesh = plsc.VectorSubcoreMesh(core_axis_name="c", subcore_axis_name="s")
    cp = pltpu.CompilerParams(needs_layout_passes=False)

    def sc_call(bbase):
        return functools.partial(
            pl.kernel,
            out_type=(
                jax.ShapeDtypeStruct((BPC * 3 * K * NQ,), jnp.float32),
                jax.ShapeDtypeStruct((BPC * K * NQ, C), jnp.float32),
            ),
            mesh=mesh,
            compiler_params=cp,
            scratch_types=[
                pltpu.VMEM((N,), jnp.float32),
                pltpu.VMEM((N,), jnp.float32),
                pltpu.VMEM((N,), jnp.float32),
                pltpu.VMEM((QPW,), jnp.float32),
                pltpu.VMEM((QPW,), jnp.float32),
                pltpu.VMEM((QPW,), jnp.float32),
                pltpu.VMEM((256,), jnp.int32),
                pltpu.VMEM((K * QPW,), jnp.int32),
                pltpu.VMEM((3 * K * QPW,), jnp.float32),
                pltpu.VMEM((CHUNK, C), jnp.float32),
                pltpu.VMEM((CHUNK, C), jnp.float32),
                pltpu.SemaphoreType.DMA,
                pltpu.SemaphoreType.DMA,
                pltpu.SemaphoreType.DMA,
            ],
        )(_make_sc_body(bbase))

    return sc_call(0), sc_call(BPC)


_sc_lo, _sc_hi = _make_calls()


SBLK = 8


def _assemble_body(gx_ref, gf_ref, o_ref):
    o_ref[0, 0:3, :] = gx_ref[0]
    for u in range(SBLK):
        o_ref[0, 3:3 + C, u * NQ:(u + 1) * NQ] = jnp.transpose(
            gf_ref[0, u], (1, 0))


def _assemble_lo(gxyz_h, gfeat_h):
    return pl.pallas_call(
        _assemble_body,
        grid=(BPC, K // SBLK),
        in_specs=[
            pl.BlockSpec((1, 3, SBLK * NQ), lambda b, s: (b, 0, s)),
            pl.BlockSpec((1, SBLK, NQ, C), lambda b, s: (b, s, 0, 0)),
        ],
        out_specs=pl.BlockSpec((1, 3 + C, SBLK * NQ), lambda b, s: (b, 0, s)),
        out_shape=jax.ShapeDtypeStruct((B, 3 + C, K * NQ), jnp.float32),
    )(gxyz_h, gfeat_h)


def _assemble_hi_body(gx_ref, gf_ref, prev_ref, o_ref):
    del prev_ref
    _assemble_body(gx_ref, gf_ref, o_ref)


def _assemble_hi(gxyz_h, gfeat_h, prev):
    return pl.pallas_call(
        _assemble_hi_body,
        grid=(BPC, K // SBLK),
        in_specs=[
            pl.BlockSpec((1, 3, SBLK * NQ), lambda b, s: (b, 0, s)),
            pl.BlockSpec((1, SBLK, NQ, C), lambda b, s: (b, s, 0, 0)),
            pl.BlockSpec(memory_space=pltpu.HBM),
        ],
        out_specs=pl.BlockSpec(
            (1, 3 + C, SBLK * NQ), lambda b, s: (BPC + b, 0, s)),
        out_shape=jax.ShapeDtypeStruct((B, 3 + C, K * NQ), jnp.float32),
        input_output_aliases={2: 0},
    )(gxyz_h, gfeat_h, prev)


@jax.jit
def kernel(xyz, new_xyz, features):
    tab = _build_table(features).reshape(B * N, C)
    xyzT = jnp.transpose(xyz, (0, 2, 1)).reshape(-1)
    newT = jnp.transpose(new_xyz, (0, 2, 1)).reshape(-1)
    gx_lo, gf_lo = _sc_lo(xyzT, newT, tab)
    gx_hi, gf_hi = _sc_hi(xyzT, newT, tab)
    out = _assemble_lo(gx_lo.reshape(BPC, 3, K * NQ),
                       gf_lo.reshape(BPC, K, NQ, C))
    out = _assemble_hi(gx_hi.reshape(BPC, 3, K * NQ),
                       gf_hi.reshape(BPC, K, NQ, C), out)
    return jnp.transpose(out.reshape(B, 3 + C, K, NQ), (0, 1, 3, 2))


# speedup vs baseline: 1.7753x; 1.2047x over previous
import functools

import jax
import jax.numpy as jnp
import numpy as np
from jax import lax
from jax.experimental import pallas as pl
from jax.experimental.pallas import tpu as pltpu
from jax.experimental.pallas import tpu_sc as plsc

RADIUS = 0.2
K = 32
R2 = np.float32(RADIUS * RADIUS)

B = 4
N = 8192
NQ = 1024
C = 128

NC = 2
NSUB = 16
L = 16
NW = NC * NSUB

WPB = NW // B
QPW = NQ // WPB
CHUNK = QPW
UNROLL = 8

SPG = K // WPB
NSTR = SPG * (NQ // CHUNK)


def _transpose_feat_body(f_ref, o_ref):
    o_ref[...] = jnp.transpose(f_ref[...], (0, 2, 1))


def _feat_transpose(features):
    return pl.pallas_call(
        _transpose_feat_body,
        grid=(B, N // 512),
        in_specs=[pl.BlockSpec((1, C, 512), lambda b, j: (b, 0, j))],
        out_specs=pl.BlockSpec((1, 512, C), lambda b, j: (b, j, 0)),
        out_shape=jax.ShapeDtypeStruct((B, N, C), jnp.float32),
    )(features)


def _scan_body(xyzT, newT, gxyz_out, idx_out,
               xpl, ypl, zpl, qx, qy, qz, buf, idxb, gxs, gys, gzs):
    cid = lax.axis_index("c")
    sid = lax.axis_index("s")
    wid = cid * NSUB + sid
    b = wid // WPB
    wq = wid % WPB
    qbase = wq * QPW

    pltpu.sync_copy(xyzT.at[pl.ds((b * 3 + 0) * N, N)], xpl)
    pltpu.sync_copy(xyzT.at[pl.ds((b * 3 + 1) * N, N)], ypl)
    pltpu.sync_copy(xyzT.at[pl.ds((b * 3 + 2) * N, N)], zpl)
    pltpu.sync_copy(newT.at[pl.ds((b * 3 + 0) * NQ + qbase, QPW)], qx)
    pltpu.sync_copy(newT.at[pl.ds((b * 3 + 1) * NQ + qbase, QPW)], qy)
    pltpu.sync_copy(newT.at[pl.ds((b * 3 + 2) * NQ + qbase, QPW)], qz)

    iota = lax.iota(jnp.int32, L)
    zeros = jnp.zeros((L,), jnp.int32)
    boffs = b * N

    def per_query(qi, carry):
        qi16 = zeros + qi
        qxv = plsc.load_gather(qx, [qi16])
        qyv = plsc.load_gather(qy, [qi16])
        qzv = plsc.load_gather(qz, [qi16])

        def cond(jc):
            j, _, cnt_s = jc
            return jnp.logical_and(j < N, cnt_s < K)

        def body(jc):
            j, csum, _ = jc
            ms = []
            for u in range(UNROLL):
                o = u * L
                dx = xpl[pl.ds(j + o, L)] - qxv
                dy = ypl[pl.ds(j + o, L)] - qyv
                dz = zpl[pl.ds(j + o, L)] - qzv
                d2 = dx * dx + dy * dy + dz * dz
                ms.append(d2 <= R2)
            for u in range(UNROLL):
                m = ms[u]
                pos = csum + plsc.cumsum(m.astype(jnp.int32)) - 1
                plsc.store_scatter(buf, [pos], j + u * L + iota, mask=m)
                csum = csum + plsc.all_reduce_population_count(m)
            return j + UNROLL * L, csum, jnp.max(csum)

        _, _, cnt = lax.while_loop(
            cond, body, (jnp.int32(0), jnp.zeros((L,), jnp.int32), jnp.int32(0)))

        blo = buf[pl.ds(0, L)]
        bhi = buf[pl.ds(L, L)]
        firstv = plsc.load_gather(buf, [zeros])
        fv = jnp.where(cnt > 0, firstv, 0)
        ilo = jnp.where(iota < cnt, blo, fv)
        ihi = jnp.where(iota + L < cnt, bhi, fv)

        rlo = iota
        rhi = iota + L
        z16 = zeros
        plsc.store_scatter(idxb, [rlo, z16, qi16], ilo + boffs)
        plsc.store_scatter(idxb, [rhi, z16, qi16], ihi + boffs)
        plsc.store_scatter(gxs, [rlo, qi16], plsc.load_gather(xpl, [ilo]) - qxv)
        plsc.store_scatter(gxs, [rhi, qi16], plsc.load_gather(xpl, [ihi]) - qxv)
        plsc.store_scatter(gys, [rlo, qi16], plsc.load_gather(ypl, [ilo]) - qyv)
        plsc.store_scatter(gys, [rhi, qi16], plsc.load_gather(ypl, [ihi]) - qyv)
        plsc.store_scatter(gzs, [rlo, qi16], plsc.load_gather(zpl, [ilo]) - qzv)
        plsc.store_scatter(gzs, [rhi, qi16], plsc.load_gather(zpl, [ihi]) - qzv)
        return carry

    lax.fori_loop(0, QPW, per_query, 0)

    pltpu.sync_copy(gxs, gxyz_out.at[b, 0, :, pl.ds(qbase, QPW)])
    pltpu.sync_copy(gys, gxyz_out.at[b, 1, :, pl.ds(qbase, QPW)])
    pltpu.sync_copy(gzs, gxyz_out.at[b, 2, :, pl.ds(qbase, QPW)])

    pltpu.sync_copy(idxb, idx_out.at[b, :, pl.ds(wq, 1), :])


KH = K // 2
SPH = KH // WPB
NSTRH = SPH * (NQ // CHUNK)


def _make_gather_body(soff):
    def _gather_body(featT, idx_in, gfeat_out, idxv, rows0, rows1, sem0, sem1):
        cid = lax.axis_index("c")
        sid = lax.axis_index("s")
        wid = cid * NSUB + sid
        b = wid // WPB
        sg = wid % WPB

        pltpu.sync_copy(idx_in.at[b, pl.ds(soff + sg * SPH, SPH), :, :], idxv)

        def issue(t, rbuf, sem):
            si = t // WPB
            qw = t % WPB
            pltpu.async_copy(featT.at[idxv.at[si, qw]], rbuf, sem)

        def drain(rbuf, sem):
            pltpu.make_async_copy(featT.at[pl.ds(0, CHUNK), :], rbuf, sem).wait()

        def writeback(t, rbuf):
            si = t // WPB
            qw = t % WPB
            roff = ((b * KH + sg * SPH + si) * WPB + qw) * CHUNK
            pltpu.sync_copy(rbuf, gfeat_out.at[pl.ds(roff, CHUNK), :])

        issue(0, rows0, sem0)

        def ring(g, carry):
            t0 = 2 * g
            t1 = 2 * g + 1
            issue(t1, rows1, sem1)
            drain(rows0, sem0)
            writeback(t0, rows0)

            @pl.when(g < NSTRH // 2 - 1)
            def _():
                issue(t0 + 2, rows0, sem0)

            drain(rows1, sem1)
            writeback(t1, rows1)
            return carry

        lax.fori_loop(0, NSTRH // 2, ring, 0)

    return _gather_body


def _make_calls():
    mesh = plsc.VectorSubcoreMesh(core_axis_name="c", subcore_axis_name="s")
    cp = pltpu.CompilerParams(needs_layout_passes=False)
    scan_call = functools.partial(
        pl.kernel,
        out_type=(
            jax.ShapeDtypeStruct((B, 3, K, NQ), jnp.float32),
            jax.ShapeDtypeStruct((B, K, WPB, QPW), jnp.int32),
        ),
        mesh=mesh,
        compiler_params=cp,
        scratch_types=[
            pltpu.VMEM((N,), jnp.float32),
            pltpu.VMEM((N,), jnp.float32),
            pltpu.VMEM((N,), jnp.float32),
            pltpu.VMEM((QPW,), jnp.float32),
            pltpu.VMEM((QPW,), jnp.float32),
            pltpu.VMEM((QPW,), jnp.float32),
            pltpu.VMEM((256,), jnp.int32),
            pltpu.VMEM((K, 1, QPW), jnp.int32),
            pltpu.VMEM((K, QPW), jnp.float32),
            pltpu.VMEM((K, QPW), jnp.float32),
            pltpu.VMEM((K, QPW), jnp.float32),
        ],
    )(_scan_body)
    def gather_half(soff):
        return functools.partial(
            pl.kernel,
            out_type=jax.ShapeDtypeStruct((B * KH * NQ, C), jnp.float32),
            mesh=mesh,
            compiler_params=cp,
            scratch_types=[
                pltpu.VMEM((SPH, WPB, QPW), jnp.int32),
                pltpu.VMEM((CHUNK, C), jnp.float32),
                pltpu.VMEM((CHUNK, C), jnp.float32),
                pltpu.SemaphoreType.DMA,
                pltpu.SemaphoreType.DMA,
            ],
        )(_make_gather_body(soff))

    return scan_call, gather_half(0), gather_half(KH)


_scan_call, _gather_lo, _gather_hi = _make_calls()


SBLK = 8


def _assemble_body(gx_ref, gf_ref, o_ref):
    o_ref[0, 0:3, :] = gx_ref[0]
    for u in range(SBLK):
        o_ref[0, 3:3 + C, u * NQ:(u + 1) * NQ] = jnp.transpose(
            gf_ref[0, u], (1, 0))


def _assemble_lo(gxyz_sc, gfeat_h):
    return pl.pallas_call(
        _assemble_body,
        grid=(B, KH // SBLK),
        in_specs=[
            pl.BlockSpec((1, 3, SBLK * NQ), lambda b, s: (b, 0, s)),
            pl.BlockSpec((1, SBLK, NQ, C), lambda b, s: (b, s, 0, 0)),
        ],
        out_specs=pl.BlockSpec((1, 3 + C, SBLK * NQ), lambda b, s: (b, 0, s)),
        out_shape=jax.ShapeDtypeStruct((B, 3 + C, K * NQ), jnp.float32),
    )(gxyz_sc, gfeat_h)


def _assemble_hi_body(gx_ref, gf_ref, prev_ref, o_ref):
    del prev_ref
    _assemble_body(gx_ref, gf_ref, o_ref)


def _assemble_hi(gxyz_sc, gfeat_h, prev):
    hb = KH // SBLK
    return pl.pallas_call(
        _assemble_hi_body,
        grid=(B, KH // SBLK),
        in_specs=[
            pl.BlockSpec((1, 3, SBLK * NQ), lambda b, s: (b, 0, hb + s)),
            pl.BlockSpec((1, SBLK, NQ, C), lambda b, s: (b, s, 0, 0)),
            pl.BlockSpec(memory_space=pltpu.HBM),
        ],
        out_specs=pl.BlockSpec((1, 3 + C, SBLK * NQ), lambda b, s: (b, 0, hb + s)),
        out_shape=jax.ShapeDtypeStruct((B, 3 + C, K * NQ), jnp.float32),
        input_output_aliases={2: 0},
    )(gxyz_sc, gfeat_h, prev)


def _assemble(gxyz_sc, gf_lo, gf_hi):
    out = _assemble_lo(gxyz_sc, gf_lo)
    out = _assemble_hi(gxyz_sc, gf_hi, out)
    return jnp.transpose(out.reshape(B, 3 + C, K, NQ), (0, 1, 3, 2))


@jax.jit
def kernel(xyz, new_xyz, features):
    featT = _feat_transpose(features).reshape(B * N, C)
    xyzT = jnp.transpose(xyz, (0, 2, 1)).reshape(-1)
    newT = jnp.transpose(new_xyz, (0, 2, 1)).reshape(-1)
    gxyz_sc, idx = _scan_call(xyzT, newT)
    gf_lo = _gather_lo(featT, idx)
    gf_hi = _gather_hi(featT, idx)
    return _assemble(gxyz_sc.reshape(B, 3, K * NQ),
                     gf_lo.reshape(B, KH, NQ, C),
                     gf_hi.reshape(B, KH, NQ, C))
